# Initial kernel scaffold; baseline (speedup 1.0000x reference)
import jax
import jax.numpy as jnp
from jax.experimental import pallas as pl


def kernel(embeddings, instance_labels):
    raise NotImplementedError("write your pallas kernel here")


# speedup vs baseline: 9.3488x; 9.3488x over previous
import jax
import jax.numpy as jnp
from jax import lax
from jax.experimental import pallas as pl
from jax.experimental.pallas import tpu as pltpu

DELTA_V = 0.5
DELTA_D = 1.5
ALPHA = 1.0
BETA = 1.0
GAMMA = 0.001
K = 64
D = 16


def _pass1_body(emb_ref, lab_ref, sums_ref, cnt_ref):
    i = pl.program_id(0)
    E = emb_ref[...]
    lab = lab_ref[0, 0, :]
    B = lab.shape[0]
    kiota = lax.broadcasted_iota(jnp.int32, (K, B), 0)
    M = (lab[None, :] == kiota).astype(jnp.float32)
    partial = jax.lax.dot(M, E, preferred_element_type=jnp.float32)
    cnt = jnp.sum(M, axis=1, keepdims=True)

    @pl.when(i == 0)
    def _():
        sums_ref[...] = jnp.zeros_like(sums_ref)
        cnt_ref[...] = jnp.zeros_like(cnt_ref)

    sums_ref[...] += partial
    cnt_ref[...] += cnt


def _pass2_body(emb_ref, lab_ref, sums_ref, cnt_ref, out_ref, mu_ref, var_ref):
    i = pl.program_id(0)
    nsteps = pl.num_programs(0)

    @pl.when(i == 0)
    def _():
        cnt = cnt_ref[...]
        mu_ref[...] = sums_ref[...] / jnp.maximum(cnt, 1.0)
        var_ref[...] = jnp.zeros_like(var_ref)

    E = emb_ref[...]
    lab = lab_ref[0, 0, :]
    B = lab.shape[0]
    kiota = lax.broadcasted_iota(jnp.int32, (B, K), 1)
    M = (lab[:, None] == kiota).astype(jnp.float32)
    C = jax.lax.dot(M, mu_ref[...], preferred_element_type=jnp.float32)
    diff = E - C
    d2 = jnp.sum(diff * diff, axis=1, keepdims=True)
    d = jnp.sqrt(d2)
    h = jnp.maximum(d - DELTA_V, 0.0)
    h2 = h * h
    var_partial = lax.dot_general(
        M, h2, (((0,), (0,)), ((), ())),
        preferred_element_type=jnp.float32)
    var_ref[...] += var_partial

    @pl.when(i == nsteps - 1)
    def _():
        cnt = cnt_ref[...]
        present = (cnt > 0.0).astype(jnp.float32)
        ni = jnp.sum(present)
        cluster_var = var_ref[...] / jnp.maximum(cnt, 1.0)
        var_loss = jnp.sum(cluster_var * present) / jnp.maximum(ni, 1.0)

        mu = mu_ref[...]
        G = lax.dot_general(mu, mu, (((1,), (1,)), ((), ())),
                            preferred_element_type=jnp.float32)
        n2 = jnp.sum(mu * mu, axis=1, keepdims=True)
        sq = jnp.maximum(n2 + n2.reshape(1, K) - 2.0 * G, 0.0)
        r = lax.broadcasted_iota(jnp.int32, (K, K), 0)
        c = lax.broadcasted_iota(jnp.int32, (K, K), 1)
        eye = r == c
        center_d = jnp.sqrt(jnp.where(eye, 1.0, sq))
        margin = 2.0 * DELTA_D
        hinged_c = jnp.maximum(margin - center_d, 0.0)
        hinged_c2 = hinged_c * hinged_c
        upper = r < c
        pair_mask = upper & (present > 0.0) & (present.reshape(1, K) > 0.0)
        num_pairs = ni * (ni - 1.0) * 0.5
        dist_loss = jnp.sum(jnp.where(pair_mask, hinged_c2, 0.0)) / jnp.maximum(num_pairs, 1.0)

        norms = jnp.sqrt(n2)
        reg_loss = jnp.sum(norms * present) / jnp.maximum(ni, 1.0)

        total = ALPHA * var_loss + BETA * dist_loss + GAMMA * reg_loss
        out_ref[...] = jnp.reshape(total, (1, 1))


def kernel(embeddings, instance_labels):
    N = embeddings.shape[0]
    B = 16000
    nsteps = N // B
    assert nsteps * B == N
    lab3 = instance_labels.astype(jnp.int32).reshape(nsteps, 1, B)

    sums, cnt = pl.pallas_call(
        _pass1_body,
        grid=(nsteps,),
        in_specs=[
            pl.BlockSpec((B, D), lambda i: (i, 0)),
            pl.BlockSpec((1, 1, B), lambda i: (i, 0, 0)),
        ],
        out_specs=[
            pl.BlockSpec((K, D), lambda i: (0, 0)),
            pl.BlockSpec((K, 1), lambda i: (0, 0)),
        ],
        out_shape=[
            jax.ShapeDtypeStruct((K, D), jnp.float32),
            jax.ShapeDtypeStruct((K, 1), jnp.float32),
        ],
        compiler_params=pltpu.CompilerParams(
            dimension_semantics=("arbitrary",)),
    )(embeddings, lab3)

    out = pl.pallas_call(
        _pass2_body,
        grid=(nsteps,),
        in_specs=[
            pl.BlockSpec((B, D), lambda i: (i, 0)),
            pl.BlockSpec((1, 1, B), lambda i: (i, 0, 0)),
            pl.BlockSpec((K, D), lambda i: (0, 0)),
            pl.BlockSpec((K, 1), lambda i: (0, 0)),
        ],
        out_specs=pl.BlockSpec((1, 1), lambda i: (0, 0)),
        out_shape=jax.ShapeDtypeStruct((1, 1), jnp.float32),
        scratch_shapes=[
            pltpu.VMEM((K, D), jnp.float32),
            pltpu.VMEM((K, 1), jnp.float32),
        ],
        compiler_params=pltpu.CompilerParams(
            dimension_semantics=("arbitrary",)),
    )(embeddings, lab3, sums, cnt)

    return out[0, 0]


# speedup vs baseline: 10.2150x; 1.0927x over previous
import jax
import jax.numpy as jnp
from jax import lax
from jax.experimental import pallas as pl
from jax.experimental.pallas import tpu as pltpu

DELTA_V = 0.5
DELTA_D = 1.5
ALPHA = 1.0
BETA = 1.0
GAMMA = 0.001
K = 64
D = 16
P = 8
KP = K * P


def _onehot_packed(labf):
    jrow = lax.broadcasted_iota(jnp.int32, (P, KP), 0)
    jcol = lax.broadcasted_iota(jnp.int32, (P, KP), 1) // K
    U = (jrow == jcol).astype(jnp.float32)
    Lrep = jax.lax.dot(labf, U, preferred_element_type=jnp.float32)
    kt = (lax.broadcasted_iota(jnp.int32, (1, KP), 1) % K).astype(jnp.float32)
    return (Lrep == kt).astype(jnp.float32)


def _pass1_body(ep_ref, lab_ref, sums_ref, cntc_ref, cntr_ref,
                s_ref, cc_ref, cr_ref):
    i = pl.program_id(0)
    nsteps = pl.num_programs(0)

    @pl.when(i == 0)
    def _():
        s_ref[...] = jnp.zeros_like(s_ref)
        cc_ref[...] = jnp.zeros_like(cc_ref)
        cr_ref[...] = jnp.zeros_like(cr_ref)

    Ep = ep_ref[...]
    labf = lab_ref[0]
    R = labf.shape[0]
    Mp = _onehot_packed(labf)
    s_ref[...] += lax.dot_general(Mp, Ep, (((0,), (0,)), ((), ())),
                                  preferred_element_type=jnp.float32)
    ones_r = jnp.ones((R, 1), jnp.float32)
    cc_ref[...] += lax.dot_general(Mp, ones_r, (((0,), (0,)), ((), ())),
                                   preferred_element_type=jnp.float32)
    cr_ref[...] += lax.dot_general(ones_r, Mp, (((0,), (0,)), ((), ())),
                                   preferred_element_type=jnp.float32)

    @pl.when(i == nsteps - 1)
    def _():
        sums = jnp.zeros((K, D), jnp.float32)
        cntc = jnp.zeros((K, 1), jnp.float32)
        cntr = jnp.zeros((1, K), jnp.float32)
        for j in range(P):
            sums = sums + s_ref[j * K:(j + 1) * K, j * D:(j + 1) * D]
            cntc = cntc + cc_ref[j * K:(j + 1) * K, :]
            cntr = cntr + cr_ref[:, j * K:(j + 1) * K]
        sums_ref[...] = sums
        cntc_ref[...] = cntc
        cntr_ref[...] = cntr


def _pass2_body(ep_ref, lab_ref, sums_ref, cntc_ref, cntr_ref, out_ref,
                w_ref, mu_ref, x_ref):
    i = pl.program_id(0)
    nsteps = pl.num_programs(0)

    @pl.when(i == 0)
    def _():
        mu = sums_ref[...] / jnp.maximum(cntc_ref[...], 1.0)
        mu_ref[...] = mu
        x_ref[...] = jnp.zeros_like(x_ref)
        w_ref[...] = jnp.zeros_like(w_ref)
        for j in range(P):
            w_ref[j * K:(j + 1) * K, j * D:(j + 1) * D] = mu

    Ep = ep_ref[...]
    labf = lab_ref[0]
    Mp = _onehot_packed(labf)
    Cp = jax.lax.dot(Mp, w_ref[...], preferred_element_type=jnp.float32)
    diff = Ep - Cp
    sp = diff * diff
    g16r = lax.broadcasted_iota(jnp.int32, (128, P), 0) // D
    g16c = lax.broadcasted_iota(jnp.int32, (128, P), 1)
    G16 = (g16r == g16c).astype(jnp.float32)
    d2p = jax.lax.dot(sp, G16, preferred_element_type=jnp.float32)
    dp = jnp.sqrt(d2p)
    hp = jnp.maximum(dp - DELTA_V, 0.0)
    hp2 = hp * hp
    x_ref[...] += lax.dot_general(hp2, Mp, (((0,), (0,)), ((), ())),
                                  preferred_element_type=jnp.float32)

    @pl.when(i == nsteps - 1)
    def _():
        var_row = jnp.zeros((1, K), jnp.float32)
        for j in range(P):
            var_row = var_row + x_ref[j:j + 1, j * K:(j + 1) * K]
        cnt_row = cntr_ref[...]
        cnt_col = cntc_ref[...]
        present_row = (cnt_row > 0.0).astype(jnp.float32)
        present_col = (cnt_col > 0.0).astype(jnp.float32)
        ni = jnp.sum(present_row)
        cluster_var = var_row / jnp.maximum(cnt_row, 1.0)
        var_loss = jnp.sum(cluster_var * present_row) / jnp.maximum(ni, 1.0)

        mu = mu_ref[...]
        G = lax.dot_general(mu, mu, (((1,), (1,)), ((), ())),
                            preferred_element_type=jnp.float32)
        r = lax.broadcasted_iota(jnp.int32, (K, K), 0)
        c = lax.broadcasted_iota(jnp.int32, (K, K), 1)
        eye = r == c
        eyef = eye.astype(jnp.float32)
        n2_col = jnp.sum(G * eyef, axis=1, keepdims=True)
        n2_row = jnp.sum(G * eyef, axis=0, keepdims=True)
        sq = jnp.maximum(n2_col + n2_row - 2.0 * G, 0.0)
        center_d = jnp.sqrt(jnp.where(eye, 1.0, sq))
        margin = 2.0 * DELTA_D
        hinged_c = jnp.maximum(margin - center_d, 0.0)
        hinged_c2 = hinged_c * hinged_c
        upper = r < c
        pair_mask = upper & (cnt_col > 0.0) & (cnt_row > 0.0)
        num_pairs = ni * (ni - 1.0) * 0.5
        dist_loss = (jnp.sum(jnp.where(pair_mask, hinged_c2, 0.0))
                     / jnp.maximum(num_pairs, 1.0))

        norms = jnp.sqrt(n2_col)
        reg_loss = jnp.sum(norms * present_col) / jnp.maximum(ni, 1.0)

        total = ALPHA * var_loss + BETA * dist_loss + GAMMA * reg_loss
        out_ref[...] = jnp.reshape(total, (1, 1))


def kernel(embeddings, instance_labels):
    N = embeddings.shape[0]
    B = 16000
    nsteps = N // B
    assert nsteps * B == N
    R = B // P
    emb_p = embeddings.reshape(N // P, P * D)
    labf = instance_labels.astype(jnp.float32).reshape(nsteps, R, P)

    sums, cntc, cntr = pl.pallas_call(
        _pass1_body,
        grid=(nsteps,),
        in_specs=[
            pl.BlockSpec((R, P * D), lambda i: (i, 0)),
            pl.BlockSpec((1, R, P), lambda i: (i, 0, 0)),
        ],
        out_specs=[
            pl.BlockSpec((K, D), lambda i: (0, 0)),
            pl.BlockSpec((K, 1), lambda i: (0, 0)),
            pl.BlockSpec((1, K), lambda i: (0, 0)),
        ],
        out_shape=[
            jax.ShapeDtypeStruct((K, D), jnp.float32),
            jax.ShapeDtypeStruct((K, 1), jnp.float32),
            jax.ShapeDtypeStruct((1, K), jnp.float32),
        ],
        scratch_shapes=[
            pltpu.VMEM((KP, P * D), jnp.float32),
            pltpu.VMEM((KP, 1), jnp.float32),
            pltpu.VMEM((1, KP), jnp.float32),
        ],
        compiler_params=pltpu.CompilerParams(
            dimension_semantics=("arbitrary",)),
    )(emb_p, labf)

    out = pl.pallas_call(
        _pass2_body,
        grid=(nsteps,),
        in_specs=[
            pl.BlockSpec((R, P * D), lambda i: (i, 0)),
            pl.BlockSpec((1, R, P), lambda i: (i, 0, 0)),
            pl.BlockSpec((K, D), lambda i: (0, 0)),
            pl.BlockSpec((K, 1), lambda i: (0, 0)),
            pl.BlockSpec((1, K), lambda i: (0, 0)),
        ],
        out_specs=pl.BlockSpec((1, 1), lambda i: (0, 0)),
        out_shape=jax.ShapeDtypeStruct((1, 1), jnp.float32),
        scratch_shapes=[
            pltpu.VMEM((KP, P * D), jnp.float32),
            pltpu.VMEM((K, D), jnp.float32),
            pltpu.VMEM((P, KP), jnp.float32),
        ],
        compiler_params=pltpu.CompilerParams(
            dimension_semantics=("arbitrary",)),
    )(emb_p, labf, sums, cntc, cntr)

    return out[0, 0]
